# Initial kernel scaffold; baseline (speedup 1.0000x reference)
import jax
import jax.numpy as jnp
from jax.experimental import pallas as pl


def kernel(words, table, mask):
    raise NotImplementedError("write your pallas kernel here")


# speedup vs baseline: 3.4750x; 3.4750x over previous
import functools

import jax
import jax.numpy as jnp
from jax import lax
from jax.experimental import pallas as pl
from jax.experimental.pallas import tpu as pltpu
from jax.experimental.pallas import tpu_sc as plsc

NUM_EMB = 100000
DIM = 64
BATCH = 4096
HIST = 50

_NC = 2
_NS = 16
_NW = _NC * _NS

_B = BATCH * HIST
_BPW = _B // _NW
_CH = 128
_NCH = _BPW // _CH
_NBUF = 5

_ROWS_BLK = 4000


def _scale_body(t_ref, m_ref, o_ref):
    o_ref[...] = t_ref[...] * m_ref[...]


def _premultiply(table, mask):
    grid = NUM_EMB // _ROWS_BLK
    return pl.pallas_call(
        _scale_body,
        grid=(grid,),
        in_specs=[
            pl.BlockSpec((_ROWS_BLK, DIM), lambda i: (i, 0)),
            pl.BlockSpec((_ROWS_BLK, 1), lambda i: (i, 0)),
        ],
        out_specs=pl.BlockSpec((_ROWS_BLK, DIM), lambda i: (i, 0)),
        out_shape=jax.ShapeDtypeStruct((NUM_EMB, DIM), jnp.float32),
    )(table, mask)


def _gather_body(idx_hbm, weight_hbm, out_hbm, idx_v, bufs, gsem, wsem):
    wid = lax.axis_index("s") * _NC + lax.axis_index("c")
    base = wid * _BPW

    pltpu.sync_copy(idx_hbm.at[wid], idx_v)

    def gather(j, b):
        pltpu.async_copy(weight_hbm.at[idx_v.at[j]], bufs.at[b], gsem.at[b])

    def write(j, b):
        pltpu.async_copy(
            bufs.at[b], out_hbm.at[pl.ds(base + j * _CH, _CH)], wsem.at[b]
        )

    for b in range(_NBUF):
        gather(b, b)

    def group(i, _):
        g = i * _NBUF
        for b in range(_NBUF):
            j = g + b
            pltpu.make_async_copy(
                weight_hbm.at[idx_v.at[j]], bufs.at[b], gsem.at[b]
            ).wait()
            write(j, b)
            pltpu.make_async_copy(
                bufs.at[b], out_hbm.at[pl.ds(base + j * _CH, _CH)], wsem.at[b]
            ).wait()
            gather(j + _NBUF, b)
        return _

    lax.fori_loop(0, _NCH // _NBUF - 1, group, 0, unroll=False)

    g = _NCH - _NBUF
    for b in range(_NBUF):
        j = g + b
        pltpu.make_async_copy(
            weight_hbm.at[idx_v.at[j]], bufs.at[b], gsem.at[b]
        ).wait()
        write(j, b)
    for b in range(_NBUF):
        j = g + b
        pltpu.make_async_copy(
            bufs.at[b], out_hbm.at[pl.ds(base + j * _CH, _CH)], wsem.at[b]
        ).wait()


@jax.jit
def _gather(idx, weight):
    mesh = plsc.VectorSubcoreMesh(core_axis_name="c", subcore_axis_name="s")
    return pl.kernel(
        _gather_body,
        mesh=mesh,
        out_type=jax.ShapeDtypeStruct((_B, DIM), jnp.float32),
        scratch_types=[
            pltpu.VMEM((_NCH, _CH), jnp.int32),
            pltpu.VMEM((_NBUF, _CH, DIM), jnp.float32),
            pltpu.SemaphoreType.DMA((_NBUF,)),
            pltpu.SemaphoreType.DMA((_NBUF,)),
        ],
        compiler_params=pltpu.CompilerParams(use_tc_tiling_on_sc=False),
    )(idx, weight)


def kernel(words, table, mask):
    weight = _premultiply(table, mask)
    idx = words.reshape(_NW, _NCH, _CH)
    out = _gather(idx, weight)
    return out.reshape(BATCH, HIST, DIM)


# speedup vs baseline: 3.9075x; 1.1244x over previous
import jax
import jax.numpy as jnp
from jax import lax
from jax.experimental import pallas as pl
from jax.experimental.pallas import tpu as pltpu
from jax.experimental.pallas import tpu_sc as plsc

NUM_EMB = 100000
DIM = 64
BATCH = 4096
HIST = 50

_NC = 2
_NS = 16
_NW = _NC * _NS

_B = BATCH * HIST
_BPW = _B // _NW
_CH = 128
_NCH = _BPW // _CH
_NBUF = 5

_ROWS2 = NUM_EMB // 2
_BLK2 = 2000


def _scale_body(t_ref, m_ref, o_ref):
    t = t_ref[...]
    m = m_ref[...]
    mb = jnp.concatenate(
        [
            jnp.broadcast_to(m[:, 0:1], (_BLK2, DIM)),
            jnp.broadcast_to(m[:, 1:2], (_BLK2, DIM)),
        ],
        axis=1,
    )
    o_ref[...] = t * mb


def _premultiply(table2, mask2):
    grid = _ROWS2 // _BLK2
    return pl.pallas_call(
        _scale_body,
        grid=(grid,),
        in_specs=[
            pl.BlockSpec((_BLK2, 2 * DIM), lambda i: (i, 0)),
            pl.BlockSpec((_BLK2, 2), lambda i: (i, 0)),
        ],
        out_specs=pl.BlockSpec((_BLK2, 2 * DIM), lambda i: (i, 0)),
        out_shape=jax.ShapeDtypeStruct((_ROWS2, 2 * DIM), jnp.float32),
    )(table2, mask2)


def _gather_body(idx_hbm, weight_hbm, out_hbm, idx_v, bufs, gsem, wsem):
    wid = lax.axis_index("s") * _NC + lax.axis_index("c")
    base = wid * _BPW

    pltpu.sync_copy(idx_hbm.at[pl.ds(base, _BPW)], idx_v)

    def idx_slice(j):
        return idx_v.at[pl.ds(j * _CH, _CH)]

    def gather(j, b):
        pltpu.async_copy(
            weight_hbm.at[idx_slice(j)], bufs.at[b], gsem.at[b]
        )

    def write(j, b):
        pltpu.async_copy(
            bufs.at[b], out_hbm.at[pl.ds(base + j * _CH, _CH)], wsem.at[b]
        )

    for b in range(_NBUF):
        gather(b, b)

    def group(i, _):
        g = i * _NBUF
        for b in range(_NBUF):
            j = g + b
            pltpu.make_async_copy(
                weight_hbm.at[idx_slice(j)], bufs.at[b], gsem.at[b]
            ).wait()
            write(j, b)
            pltpu.make_async_copy(
                bufs.at[b], out_hbm.at[pl.ds(base + j * _CH, _CH)], wsem.at[b]
            ).wait()
            gather(j + _NBUF, b)
        return _

    lax.fori_loop(0, _NCH // _NBUF - 1, group, 0, unroll=False)

    g = _NCH - _NBUF
    for b in range(_NBUF):
        j = g + b
        pltpu.make_async_copy(
            weight_hbm.at[idx_slice(j)], bufs.at[b], gsem.at[b]
        ).wait()
        write(j, b)
    for b in range(_NBUF):
        j = g + b
        pltpu.make_async_copy(
            bufs.at[b], out_hbm.at[pl.ds(base + j * _CH, _CH)], wsem.at[b]
        ).wait()


@jax.jit
def _gather(idx, weight):
    mesh = plsc.VectorSubcoreMesh(core_axis_name="c", subcore_axis_name="s")
    return pl.kernel(
        _gather_body,
        mesh=mesh,
        out_type=jax.ShapeDtypeStruct((_B, DIM), jnp.float32),
        scratch_types=[
            pltpu.VMEM((_BPW,), jnp.int32),
            pltpu.VMEM((_NBUF, _CH, DIM), jnp.float32),
            pltpu.SemaphoreType.DMA((_NBUF,)),
            pltpu.SemaphoreType.DMA((_NBUF,)),
        ],
        compiler_params=pltpu.CompilerParams(use_tc_tiling_on_sc=False),
    )(idx, weight)


def kernel(words, table, mask):
    table2 = table.reshape(_ROWS2, 2 * DIM)
    mask2 = mask.reshape(_ROWS2, 2)
    weight = _premultiply(table2, mask2).reshape(NUM_EMB, DIM)
    idx = words.reshape(_B)
    out = _gather(idx, weight)
    return out.reshape(BATCH, HIST, DIM)


# speedup vs baseline: 4.1474x; 1.0614x over previous
import jax
import jax.numpy as jnp
import numpy as np
from jax import lax
from jax.experimental import pallas as pl
from jax.experimental.pallas import tpu as pltpu
from jax.experimental.pallas import tpu_sc as plsc

NUM_EMB = 100000
DIM = 64
BATCH = 4096
HIST = 50

_NC = 2
_NS = 16
_NW = _NC * _NS

_B = BATCH * HIST
_BPW = _B // _NW
_CH = 128
_NCH = _BPW // _CH
_NBUF = 5
_L = 16


def _gather_body(idx_hbm, table_hbm, mask_hbm, out_hbm,
                 idx_v, bufs, mbufs, gsem, msem, wsem):
    wid = lax.axis_index("s") * _NC + lax.axis_index("c")
    base = wid * _BPW

    pltpu.sync_copy(idx_hbm.at[pl.ds(base, _BPW)], idx_v)

    def idx_slice(j):
        return idx_v.at[pl.ds(j * _CH, _CH)]

    def gather(j, b):
        pltpu.async_copy(table_hbm.at[idx_slice(j)], bufs.at[b], gsem.at[b])
        pltpu.async_copy(mask_hbm.at[idx_slice(j)], mbufs.at[b], msem.at[b])

    def wait_gather(j, b):
        pltpu.make_async_copy(
            table_hbm.at[idx_slice(j)], bufs.at[b], gsem.at[b]
        ).wait()
        pltpu.make_async_copy(
            mask_hbm.at[idx_slice(j)], mbufs.at[b], msem.at[b]
        ).wait()

    def write(j, b):
        pltpu.async_copy(
            bufs.at[b], out_hbm.at[pl.ds(base + j * _CH, _CH)], wsem.at[b]
        )

    def wait_write(j, b):
        pltpu.make_async_copy(
            bufs.at[b], out_hbm.at[pl.ds(base + j * _CH, _CH)], wsem.at[b]
        ).wait()

    lane_const = [jnp.full((_L,), r, jnp.int32) for r in range(_L)]

    def multiply(b):
        def grp(g, _):
            m16 = mbufs[b, pl.ds(g * _L, _L)]
            for r in range(_L):
                mval = jnp.take(m16, lane_const[r])
                row = g * _L + r
                for d in range(DIM // _L):
                    sl = pl.ds(d * _L, _L)
                    bufs[b, row, sl] = bufs[b, row, sl] * mval
            return _

        lax.fori_loop(0, _CH // _L, grp, 0, unroll=False)

    for b in range(_NBUF):
        gather(b, b)

    def group(i, _):
        g = i * _NBUF
        for b in range(_NBUF):
            j = g + b
            wait_gather(j, b)
            multiply(b)
            write(j, b)
            wait_write(j, b)
            gather(j + _NBUF, b)
        return _

    lax.fori_loop(0, _NCH // _NBUF - 1, group, 0, unroll=False)

    g = _NCH - _NBUF
    for b in range(_NBUF):
        j = g + b
        wait_gather(j, b)
        multiply(b)
        write(j, b)
    for b in range(_NBUF):
        wait_write(g + b, b)


@jax.jit
def _gather(idx, table, mask1):
    mesh = plsc.VectorSubcoreMesh(core_axis_name="c", subcore_axis_name="s")
    return pl.kernel(
        _gather_body,
        mesh=mesh,
        out_type=jax.ShapeDtypeStruct((_B, DIM), jnp.float32),
        scratch_types=[
            pltpu.VMEM((_BPW,), jnp.int32),
            pltpu.VMEM((_NBUF, _CH, DIM), jnp.float32),
            pltpu.VMEM((_NBUF, _CH), jnp.float32),
            pltpu.SemaphoreType.DMA((_NBUF,)),
            pltpu.SemaphoreType.DMA((_NBUF,)),
            pltpu.SemaphoreType.DMA((_NBUF,)),
        ],
        compiler_params=pltpu.CompilerParams(use_tc_tiling_on_sc=False),
    )(idx, table, mask1)


def kernel(words, table, mask):
    idx = words.reshape(_B)
    mask1 = mask.reshape(NUM_EMB)
    out = _gather(idx, table, mask1)
    return out.reshape(BATCH, HIST, DIM)
